# baseline (speedup 1.0000x reference)
import jax
import jax.numpy as jnp
from jax.experimental import pallas as pl

_E = 800000
_T = 8
_C = 8
_B = 2048


def _tc_body(it_ref, oc_ref, tt_ref, ct_ref, lut_ref, td_ref, cd_ref, out_ref):
    it = it_ref[:]
    oc = oc_ref[:]
    tt = tt_ref[:, :]
    ct = ct_ref[:, :]
    lut = lut_ref[:, :]
    td = td_ref[:]
    cd = cd_ref[:]

    eps = jnp.float32(1e-12)

    t_idx = jnp.sum((tt <= it[:, None]).astype(jnp.int32), axis=1)
    c_idx = jnp.sum((ct <= oc[:, None]).astype(jnp.int32), axis=1)

    max_t = jnp.maximum(td - 1, 0)
    max_c = jnp.maximum(cd - 1, 0)
    t_hi = jnp.minimum(jnp.maximum(t_idx, 1), max_t)
    c_hi = jnp.minimum(jnp.maximum(c_idx, 1), max_c)
    t_lo = jnp.maximum(t_hi - 1, 0)
    c_lo = jnp.maximum(c_hi - 1, 0)

    iota8 = jax.lax.broadcasted_iota(jnp.int32, tt.shape, 1)
    zero8 = jnp.zeros_like(tt)
    t0 = jnp.sum(jnp.where(iota8 == t_lo[:, None], tt, zero8), axis=1)
    t1 = jnp.sum(jnp.where(iota8 == t_hi[:, None], tt, zero8), axis=1)
    c0 = jnp.sum(jnp.where(iota8 == c_lo[:, None], ct, zero8), axis=1)
    c1 = jnp.sum(jnp.where(iota8 == c_hi[:, None], ct, zero8), axis=1)

    stride = cd
    idx00 = t_lo * stride + c_lo
    idx01 = t_lo * stride + c_hi
    idx10 = t_hi * stride + c_lo
    idx11 = t_hi * stride + c_hi

    iota64 = jax.lax.broadcasted_iota(jnp.int32, lut.shape, 1)
    zero64 = jnp.zeros_like(lut)
    v00 = jnp.sum(jnp.where(iota64 == idx00[:, None], lut, zero64), axis=1)
    v01 = jnp.sum(jnp.where(iota64 == idx01[:, None], lut, zero64), axis=1)
    v10 = jnp.sum(jnp.where(iota64 == idx10[:, None], lut, zero64), axis=1)
    v11 = jnp.sum(jnp.where(iota64 == idx11[:, None], lut, zero64), axis=1)

    t_interval = t1 - t0
    c_interval = c1 - c0
    is_t_deg = jnp.abs(t_interval) < eps
    is_c_deg = jnp.abs(c_interval) < eps
    x = jnp.clip(it, t0, t1)
    y = jnp.clip(oc, c0, c1)
    t_safe = jnp.where(is_t_deg, eps, t_interval)
    c_safe = jnp.where(is_c_deg, eps, c_interval)
    denom = t_safe * c_safe
    wa = (t1 - x) * (c1 - y)
    wb = (t1 - x) * (y - c0)
    wc = (x - t0) * (c1 - y)
    wd = (x - t0) * (y - c0)
    bilinear = (v00 * wa + v01 * wb + v10 * wc + v11 * wd) / denom
    fc = jnp.clip((y - c0) / c_safe, 0.0, 1.0)
    ft = jnp.clip((x - t0) / t_safe, 0.0, 1.0)
    lerp_c = v00 + fc * (v01 - v00)
    lerp_t = v00 + ft * (v10 - v00)
    out_ref[:] = jnp.where(
        is_t_deg & is_c_deg, v00,
        jnp.where(is_t_deg, lerp_c, jnp.where(is_c_deg, lerp_t, bilinear)))


def kernel(input_trans, output_caps, trans_tables, cap_tables, lut_values, trans_dims, cap_dims):
    grid = (pl.cdiv(_E, _B),)
    return pl.pallas_call(
        _tc_body,
        grid=grid,
        in_specs=[
            pl.BlockSpec((_B,), lambda i: (i,)),
            pl.BlockSpec((_B,), lambda i: (i,)),
            pl.BlockSpec((_B, _T), lambda i: (i, 0)),
            pl.BlockSpec((_B, _C), lambda i: (i, 0)),
            pl.BlockSpec((_B, _T * _C), lambda i: (i, 0)),
            pl.BlockSpec((_B,), lambda i: (i,)),
            pl.BlockSpec((_B,), lambda i: (i,)),
        ],
        out_specs=pl.BlockSpec((_B,), lambda i: (i,)),
        out_shape=jax.ShapeDtypeStruct((_E,), jnp.float32),
    )(input_trans, output_caps, trans_tables, cap_tables, lut_values,
      trans_dims, cap_dims)


# speedup vs baseline: 3.4103x; 3.4103x over previous
import functools

import jax
import jax.numpy as jnp
from jax import lax
from jax.experimental import pallas as pl
from jax.experimental.pallas import tpu as pltpu
from jax.experimental.pallas import tpu_sc as plsc

_E = 800000
_T = 8
_C = 8
_L = 16
_NW = 32
_CH = 1280
_NCHUNK = _E // _CH
_MAXIT = -(-_NCHUNK // _NW)
_G = _CH // _L


def _sc_body(it_h, oc_h, tt_h, ct_h, lut_h, td_h, cd_h, out_h,
             it_v, oc_v, tt_v, ct_v, td_v, cd_v,
             i00_v, i01_v, i10_v, i11_v,
             a00_v, a01_v, a10_v, a11_v,
             v00_v, v01_v, v10_v, v11_v,
             out_v, sem):
    wid = lax.axis_index("s") * 2 + lax.axis_index("c")
    lane = jnp.arange(_L, dtype=jnp.int32)
    eps = jnp.float32(1e-12)

    def chunk_body(i, carry):
        c = wid + i * _NW

        @pl.when(c < _NCHUNK)
        def _():
            base = c * _CH
            cps = [
                pltpu.async_copy(it_h.at[pl.ds(base, _CH)], it_v, sem),
                pltpu.async_copy(oc_h.at[pl.ds(base, _CH)], oc_v, sem),
                pltpu.async_copy(tt_h.at[pl.ds(base * _T, _CH * _T)], tt_v, sem),
                pltpu.async_copy(ct_h.at[pl.ds(base * _C, _CH * _C)], ct_v, sem),
                pltpu.async_copy(td_h.at[pl.ds(base, _CH)], td_v, sem),
                pltpu.async_copy(cd_h.at[pl.ds(base, _CH)], cd_v, sem),
            ]
            for cp in cps:
                cp.wait()

            def g_body(g, carry2):
                s = g * _L
                rows = s + lane
                it = it_v[pl.ds(s, _L)]
                oc = oc_v[pl.ds(s, _L)]
                td = td_v[pl.ds(s, _L)]
                cd = cd_v[pl.ds(s, _L)]

                rows8 = rows * _T
                t_idx = jnp.zeros((_L,), jnp.int32)
                c_idx = jnp.zeros((_L,), jnp.int32)
                for j in range(_T):
                    ttj = plsc.load_gather(tt_v, [rows8 + j])
                    ctj = plsc.load_gather(ct_v, [rows8 + j])
                    t_idx = t_idx + (ttj <= it).astype(jnp.int32)
                    c_idx = c_idx + (ctj <= oc).astype(jnp.int32)

                max_t = jnp.maximum(td - 1, 0)
                max_c = jnp.maximum(cd - 1, 0)
                t_hi = jnp.minimum(jnp.maximum(t_idx, 1), max_t)
                c_hi = jnp.minimum(jnp.maximum(c_idx, 1), max_c)
                t_lo = t_hi - 1
                c_lo = c_hi - 1

                t0 = plsc.load_gather(tt_v, [rows8 + t_lo])
                t1 = plsc.load_gather(tt_v, [rows8 + t_hi])
                c0 = plsc.load_gather(ct_v, [rows8 + c_lo])
                c1 = plsc.load_gather(ct_v, [rows8 + c_hi])

                arc = base + rows
                i00 = arc * (_T * _C) + t_lo * cd + c_lo
                i10 = i00 + cd

                t_int = t1 - t0
                c_int = c1 - c0
                t_deg = jnp.abs(t_int) < eps
                c_deg = jnp.abs(c_int) < eps
                x = jnp.clip(it, t0, t1)
                y = jnp.clip(oc, c0, c1)
                ts = jnp.where(t_deg, eps, t_int)
                cs = jnp.where(c_deg, eps, c_int)
                rt = jnp.float32(1.0) / ts
                rc = jnp.float32(1.0) / cs
                rd = rt * rc
                dx0 = x - t0
                dx1 = t1 - x
                dy0 = y - c0
                dy1 = c1 - y
                b00 = dx1 * dy1 * rd
                b01 = dx1 * dy0 * rd
                b10 = dx0 * dy1 * rd
                b11 = dx0 * dy0 * rd
                fc = jnp.clip(dy0 * rc, 0.0, 1.0)
                ft = jnp.clip(dx0 * rt, 0.0, 1.0)
                one = jnp.float32(1.0)
                zero = jnp.float32(0.0)
                a00 = jnp.where(t_deg, jnp.where(c_deg, one, one - fc),
                                jnp.where(c_deg, one - ft, b00))
                a01 = jnp.where(t_deg, jnp.where(c_deg, zero, fc),
                                jnp.where(c_deg, zero, b01))
                a10 = jnp.where(t_deg, zero, jnp.where(c_deg, ft, b10))
                a11 = jnp.where(t_deg, zero, jnp.where(c_deg, zero, b11))

                i00_v[pl.ds(s, _L)] = i00
                i01_v[pl.ds(s, _L)] = i00 + 1
                i10_v[pl.ds(s, _L)] = i10
                i11_v[pl.ds(s, _L)] = i10 + 1
                a00_v[pl.ds(s, _L)] = a00
                a01_v[pl.ds(s, _L)] = a01
                a10_v[pl.ds(s, _L)] = a10
                a11_v[pl.ds(s, _L)] = a11
                return carry2

            lax.fori_loop(0, _G, g_body, 0)

            gs = [
                pltpu.async_copy(lut_h.at[i00_v], v00_v, sem),
                pltpu.async_copy(lut_h.at[i01_v], v01_v, sem),
                pltpu.async_copy(lut_h.at[i10_v], v10_v, sem),
                pltpu.async_copy(lut_h.at[i11_v], v11_v, sem),
            ]
            for cp in gs:
                cp.wait()

            def h_body(g, carry2):
                s = g * _L
                d = pl.ds(s, _L)
                out_v[d] = (a00_v[d] * v00_v[d] + a01_v[d] * v01_v[d]
                            + a10_v[d] * v10_v[d] + a11_v[d] * v11_v[d])
                return carry2

            lax.fori_loop(0, _G, h_body, 0)
            pltpu.sync_copy(out_v, out_h.at[pl.ds(base, _CH)])

        return carry

    lax.fori_loop(0, _MAXIT, chunk_body, 0)


@jax.jit
def _sc_call(input_trans, output_caps, trans_tables, cap_tables, lut_flat,
             trans_dims, cap_dims):
    mesh = plsc.VectorSubcoreMesh(core_axis_name="c", subcore_axis_name="s")
    f = pl.kernel(
        _sc_body,
        out_type=jax.ShapeDtypeStruct((_E,), jnp.float32),
        mesh=mesh,
        compiler_params=pltpu.CompilerParams(needs_layout_passes=False),
        scratch_types=[
            pltpu.VMEM((_CH,), jnp.float32),
            pltpu.VMEM((_CH,), jnp.float32),
            pltpu.VMEM((_CH * _T,), jnp.float32),
            pltpu.VMEM((_CH * _C,), jnp.float32),
            pltpu.VMEM((_CH,), jnp.int32),
            pltpu.VMEM((_CH,), jnp.int32),
            pltpu.VMEM((_CH,), jnp.int32),
            pltpu.VMEM((_CH,), jnp.int32),
            pltpu.VMEM((_CH,), jnp.int32),
            pltpu.VMEM((_CH,), jnp.int32),
            pltpu.VMEM((_CH,), jnp.float32),
            pltpu.VMEM((_CH,), jnp.float32),
            pltpu.VMEM((_CH,), jnp.float32),
            pltpu.VMEM((_CH,), jnp.float32),
            pltpu.VMEM((_CH,), jnp.float32),
            pltpu.VMEM((_CH,), jnp.float32),
            pltpu.VMEM((_CH,), jnp.float32),
            pltpu.VMEM((_CH,), jnp.float32),
            pltpu.VMEM((_CH,), jnp.float32),
            pltpu.SemaphoreType.DMA,
        ],
    )
    return f(input_trans, output_caps, trans_tables, cap_tables, lut_flat,
             trans_dims, cap_dims)


def kernel(input_trans, output_caps, trans_tables, cap_tables, lut_values,
           trans_dims, cap_dims):
    lut_flat = lut_values.reshape(_E * _T * _C)
    tt_flat = trans_tables.reshape(_E * _T)
    ct_flat = cap_tables.reshape(_E * _C)
    return _sc_call(input_trans, output_caps, tt_flat, ct_flat,
                    lut_flat, trans_dims, cap_dims)
